# baseline (speedup 1.0000x reference)
import functools

import jax
import jax.numpy as jnp
from jax import lax
from jax.experimental import pallas as pl
from jax.experimental.pallas import tpu as pltpu
from jax.experimental.pallas import tpu_sc as plsc

B = 16384
N = 100000
D = 64
L = 16

_info = plsc.get_sparse_core_info()
NC, NS = _info.num_cores, _info.num_subcores
BH = B // NC
DPS = D // NS
BQ = BH // 2
OPS = BH // NS


def _sc_body(i_hbm, j_hbm, k_hbm, wt_hbm, vt_hbm, ut_hbm, out_hbm,
             col_v, idxq, selv, selu, prod, rows_v, partial):
    c = lax.axis_index("c")
    s = lax.axis_index("s")
    bbase = c * BH
    lane = jnp.arange(L, dtype=jnp.int32)

    for g in range(4):
        for u in range(8):
            prod[g, pl.ds(u * L, L)] = jnp.zeros((L,), jnp.float32)
    pltpu.sync_copy(prod.at[pl.ds(0, 4), :], partial.at[pl.ds(s * 4, 4), :])
    plsc.subcore_barrier()

    def gather_into(dst, n):
        def blk(g, carry):
            for u in range(4):
                o = (g * 4 + u) * L
                iv = idxq[pl.ds(o, L)]
                dst[pl.ds(o, L)] = plsc.load_gather(col_v, [iv])
            return carry

        lax.fori_loop(0, n // L // 4, blk, 0)

    for dd in range(DPS):
        d = s * DPS + dd

        pltpu.sync_copy(j_hbm.at[pl.ds(bbase, BH)], idxq)
        pltpu.sync_copy(vt_hbm.at[d], col_v)
        gather_into(selv, BH)

        pltpu.sync_copy(k_hbm.at[pl.ds(bbase, BH)], idxq)
        pltpu.sync_copy(ut_hbm.at[d], col_v)
        gather_into(selu, BH)

        pltpu.sync_copy(i_hbm.at[pl.ds(bbase, BH)], idxq)
        pltpu.sync_copy(wt_hbm.at[d], col_v)
        for q in range(BH // BQ):
            rows_v[pl.ds(0, L)] = lane + q * 32
            rows_v[pl.ds(L, L)] = lane + q * 32 + L

            def blk(g, carry, q=q):
                for u in range(8):
                    o = q * BQ + g * 128 + u * L
                    iv = idxq[pl.ds(o, L)]
                    wv = plsc.load_gather(col_v, [iv])
                    prod[g, pl.ds(u * L, L)] = (wv * selv[pl.ds(o, L)]
                                                * selu[pl.ds(o, L)])
                return carry

            lax.fori_loop(0, BQ // 128, blk, 0)
            pltpu.sync_copy(prod, partial.at[rows_v], add=True)

    plsc.subcore_barrier()

    pltpu.sync_copy(partial.at[pl.ds(s * 4, 4), :], prod.at[pl.ds(0, 4), :])
    for g in range(4):
        for u in range(8):
            sl = pl.ds(u * L, L)
            selv[pl.ds(g * 128 + u * L, L)] = (
                1.0 / (1.0 + jnp.exp(-prod[g, sl])))
    pltpu.sync_copy(selv.at[pl.ds(0, OPS)],
                    out_hbm.at[pl.ds(bbase + s * OPS, OPS)])


@functools.partial(jax.jit, static_argnums=())
def kernel(indices, W, V, U):
    idx = indices.astype(jnp.int32)
    i_idx, j_idx, k_idx = idx[:, 0], idx[:, 1], idx[:, 2]
    WT, VT, UT = W.T, V.T, U.T

    mesh = plsc.VectorSubcoreMesh(core_axis_name="c", subcore_axis_name="s")
    run = pl.kernel(
        _sc_body,
        mesh=mesh,
        out_type=jax.ShapeDtypeStruct((B,), jnp.float32),
        scratch_types=[
            pltpu.VMEM((N,), jnp.float32),
            pltpu.VMEM((BH,), jnp.int32),
            pltpu.VMEM((BH,), jnp.float32),
            pltpu.VMEM((BH,), jnp.float32),
            pltpu.VMEM((BQ // 128, 128), jnp.float32),
            pltpu.VMEM((32,), jnp.int32),
            pltpu.VMEM_SHARED((BH // 128, 128), jnp.float32),
        ],
        compiler_params=pltpu.CompilerParams(needs_layout_passes=False,
                                             use_tc_tiling_on_sc=False),
    )
    return run(i_idx, j_idx, k_idx, WT, VT, UT)


# speedup vs baseline: 1.1667x; 1.1667x over previous
import functools

import jax
import jax.numpy as jnp
from jax import lax
from jax.experimental import pallas as pl
from jax.experimental.pallas import tpu as pltpu
from jax.experimental.pallas import tpu_sc as plsc

B = 16384
D = 64
L = 16

_info = plsc.get_sparse_core_info()
NC, NS = _info.num_cores, _info.num_subcores
NW = NC * NS
BPW = B // NW
NCHUNK = 4
CHUNK = BPW // NCHUNK
NBLK = BPW // L


def _sc_body(idx_hbm, w_hbm, v_hbm, u_hbm, out_hbm,
             idx_v, w_rows, v_rows, u_rows, out_v, sem):
    wid = lax.axis_index("s") * NC + lax.axis_index("c")

    pltpu.sync_copy(idx_hbm.at[wid], idx_v)

    handles = []
    for t, (tab, rows) in enumerate(
            ((w_hbm, w_rows), (v_hbm, v_rows), (u_hbm, u_rows))):
        for c in range(NCHUNK):
            handles.append(pltpu.async_copy(
                tab.at[idx_v.at[t, c]],
                rows.at[pl.ds(c * CHUNK, CHUNK), :],
                sem))
    for h in handles:
        h.wait()

    lane = jnp.arange(L, dtype=jnp.int32)

    def blk_body(blk, carry):
        base = blk * L
        thetas = jnp.zeros((L,), jnp.float32)
        for r in range(L):
            row = base + r
            acc = jnp.zeros((L,), jnp.float32)
            for c in range(D // L):
                sl = pl.ds(c * L, L)
                acc = acc + w_rows[row, sl] * v_rows[row, sl] * u_rows[row, sl]
            theta = jnp.sum(acc)
            thetas = thetas + jnp.where(lane == r, theta, 0.0)
        probs = 1.0 / (1.0 + jnp.exp(-thetas))
        out_v[pl.ds(base, L)] = probs
        return carry

    lax.fori_loop(0, NBLK, blk_body, 0)

    pltpu.sync_copy(out_v, out_hbm.at[pl.ds(wid * BPW, BPW)])


@functools.partial(jax.jit, static_argnums=())
def kernel(indices, W, V, U):
    idx = indices.astype(jnp.int32).T
    idx = idx.reshape(3, NW, NCHUNK, CHUNK).transpose(1, 0, 2, 3)

    mesh = plsc.VectorSubcoreMesh(core_axis_name="c", subcore_axis_name="s")
    run = pl.kernel(
        _sc_body,
        mesh=mesh,
        out_type=jax.ShapeDtypeStruct((B,), jnp.float32),
        scratch_types=[
            pltpu.VMEM((3, NCHUNK, CHUNK), jnp.int32),
            pltpu.VMEM((BPW, D), jnp.float32),
            pltpu.VMEM((BPW, D), jnp.float32),
            pltpu.VMEM((BPW, D), jnp.float32),
            pltpu.VMEM((BPW,), jnp.float32),
            pltpu.SemaphoreType.DMA,
        ],
        compiler_params=pltpu.CompilerParams(
            needs_layout_passes=False, use_tc_tiling_on_sc=False),
    )
    return run(idx, W, V, U)
